# Initial kernel scaffold; baseline (speedup 1.0000x reference)
import jax
import jax.numpy as jnp
from jax.experimental import pallas as pl


def kernel(utterance, table, data_grad):
    raise NotImplementedError("write your pallas kernel here")


# speedup vs baseline: 2.6282x; 2.6282x over previous
import functools

import jax
import jax.numpy as jnp
from jax import lax
from jax.experimental import pallas as pl
from jax.experimental.pallas import tpu as pltpu
from jax.experimental.pallas import tpu_sc as plsc

B = 16
L = 50
V = 100000
D = 128
EPS = 0.4
K_SEL = 25
STD_MULT = 3.0
KNN_K = 8

Q = B * L
V_TILE = 2048
N_TILES = -(-V // V_TILE)
V_PAD = N_TILES * V_TILE

NEG_INF = float("-inf")
BIG_I32 = 2147483647

_NC = 2
_NS = 16
_NW = _NC * _NS
Q_PAD = 1024
_BPW = Q_PAD // _NW


def _sc_gather_body(table_hbm, idx_hbm, out_hbm, idx_v, rows_v, sem):
    wid = lax.axis_index("s") * _NC + lax.axis_index("c")
    base = wid * _BPW
    pltpu.sync_copy(idx_hbm.at[pl.ds(base, _BPW)], idx_v)
    pltpu.async_copy(table_hbm.at[idx_v], rows_v, sem).wait()
    pltpu.sync_copy(rows_v, out_hbm.at[pl.ds(base, _BPW)])


def _gather_rows(table, idx_pad):
    mesh = plsc.VectorSubcoreMesh(
        core_axis_name="c", subcore_axis_name="s",
        num_cores=_NC, num_subcores=_NS)
    k = pl.kernel(
        _sc_gather_body,
        out_type=jax.ShapeDtypeStruct((Q_PAD, D), jnp.float32),
        mesh=mesh,
        scratch_types=[
            pltpu.VMEM((_BPW,), jnp.int32),
            pltpu.VMEM((_BPW, D), jnp.float32),
            pltpu.SemaphoreType.DMA,
        ],
    )
    return k(table, idx_pad)


def _fgsm_pert(in3, dg3):
    absg = jnp.broadcast_to(
        jnp.sum(jnp.abs(dg3), axis=-1, keepdims=True), (B, L, D))
    l_iota = lax.broadcasted_iota(jnp.int32, (B, L, D), 1)
    rank = jnp.zeros((B, L, D), jnp.int32)
    for m in range(L):
        am = absg[:, m:m + 1, :]
        beats = (am > absg) | ((am == absg) & (m < l_iota))
        rank = rank + beats.astype(jnp.int32)
    mask3 = rank < K_SEL
    cnt = float(B * K_SEL * D)
    g1 = jnp.where(mask3, dg3, 0.0)
    mean = jnp.sum(g1) / cnt
    var = jnp.sum(jnp.where(mask3, (dg3 - mean) ** 2, 0.0)) / (cnt - 1.0)
    std = jnp.sqrt(var)
    lower = mean - std * STD_MULT
    upper = mean + std * STD_MULT
    mask2 = mask3 & ((dg3 < lower) | (dg3 > upper))
    sign = jnp.sign(jnp.where(mask2, dg3, 0.0))
    pert3 = in3 + EPS * sign
    return pert3.reshape(Q, D)


def _knn_body(tT_ref, in_ref, dg_ref, vals_ref, idx_ref, pert_ref, qn_ref):
    i = pl.program_id(0)

    @pl.when(i == 0)
    def _init():
        pert = _fgsm_pert(in_ref[...], dg_ref[...])
        pert_ref[...] = pert
        qn_ref[...] = jnp.sqrt(jnp.sum(pert * pert, axis=1, keepdims=True))
        vals_ref[...] = jnp.full((Q, KNN_K), NEG_INF, jnp.float32)
        idx_ref[...] = (2 ** 30
                        + lax.broadcasted_iota(jnp.int32, (Q, KNN_K), 1))

    tt = tT_ref[...]
    pert = pert_ref[...]
    s = lax.dot_general(pert, tt, (((1,), (0,)), ((), ())),
                        preferred_element_type=jnp.float32)
    tn = jnp.sqrt(jnp.sum(tt * tt, axis=0, keepdims=True))
    denom = jnp.maximum(qn_ref[...] * tn, 1e-8)
    cos = s / denom
    col = i * V_TILE + lax.broadcasted_iota(jnp.int32, (Q, V_TILE), 1)
    cos = jnp.where(col < V, cos, NEG_INF)

    tvals, tidx = [], []
    work = cos
    for _ in range(KNN_K):
        m = jnp.max(work, axis=1, keepdims=True)
        mi = jnp.min(jnp.where(work == m, col, BIG_I32), axis=1, keepdims=True)
        tvals.append(m)
        tidx.append(mi)
        work = jnp.where(col == mi, NEG_INF, work)

    cv = jnp.concatenate([vals_ref[...]] + tvals, axis=1)
    ci = jnp.concatenate([idx_ref[...]] + tidx, axis=1)
    ov, oi = [], []
    for _ in range(KNN_K):
        m = jnp.max(cv, axis=1, keepdims=True)
        mi = jnp.min(jnp.where(cv == m, ci, BIG_I32), axis=1, keepdims=True)
        ov.append(m)
        oi.append(mi)
        cv = jnp.where((cv == m) & (ci == mi), NEG_INF, cv)
    vals_ref[...] = jnp.concatenate(ov, axis=1)
    idx_ref[...] = jnp.concatenate(oi, axis=1)


def _knn_topk(tT, in3, dg3):
    grid = (N_TILES,)
    return pl.pallas_call(
        _knn_body,
        grid=grid,
        in_specs=[
            pl.BlockSpec((D, V_TILE), lambda i: (0, i)),
            pl.BlockSpec((B, L, D), lambda i: (0, 0, 0)),
            pl.BlockSpec((B, L, D), lambda i: (0, 0, 0)),
        ],
        out_specs=[
            pl.BlockSpec((Q, KNN_K), lambda i: (0, 0)),
            pl.BlockSpec((Q, KNN_K), lambda i: (0, 0)),
        ],
        out_shape=[
            jax.ShapeDtypeStruct((Q, KNN_K), jnp.float32),
            jax.ShapeDtypeStruct((Q, KNN_K), jnp.int32),
        ],
        scratch_shapes=[
            pltpu.VMEM((Q, D), jnp.float32),
            pltpu.VMEM((Q, 1), jnp.float32),
        ],
    )(tT, in3, dg3)


def kernel(utterance, table, data_grad):
    idx = utterance.reshape(-1).astype(jnp.int32)
    idx_pad = jnp.concatenate([idx, jnp.zeros((Q_PAD - Q,), jnp.int32)])
    rows = _gather_rows(table, idx_pad)
    in3 = rows[:Q].reshape(B, L, D)
    tT = jnp.pad(table, ((0, V_PAD - V), (0, 0))).T
    vals, idx8 = _knn_topk(tT, in3, data_grad)
    return vals.reshape(B, L, KNN_K), idx8.reshape(B, L, KNN_K)


# speedup vs baseline: 4.2531x; 1.6183x over previous
import jax
import jax.numpy as jnp
from jax import lax
from jax.experimental import pallas as pl
from jax.experimental.pallas import tpu as pltpu
from jax.experimental.pallas import tpu_sc as plsc

B = 16
L = 50
V = 100000
D = 128
EPS = 0.4
K_SEL = 25
STD_MULT = 3.0
KNN_K = 8

Q = B * L
V_TILE = 2048
N_TILES = -(-V // V_TILE)
V_PAD = N_TILES * V_TILE

N_CHUNK = 9
N_CAND = N_CHUNK * N_TILES

NEG_INF = float("-inf")
BIG_I32 = 2147483647

_NC = 2
_NS = 16
_NW = _NC * _NS
Q_PAD = 1024
_BPW = Q_PAD // _NW

CAND_BATCH = 128
CAND_TOT = 356352
CAND_PW = CAND_TOT // _NW
CAND_NB = CAND_PW // CAND_BATCH


def _sc_gather_body(table_hbm, idx_hbm, out_hbm, idx_v, rows_v, sem):
    wid = lax.axis_index("s") * _NC + lax.axis_index("c")
    base = wid * _BPW
    pltpu.sync_copy(idx_hbm.at[pl.ds(base, _BPW)], idx_v)
    pltpu.async_copy(table_hbm.at[idx_v], rows_v, sem).wait()
    pltpu.sync_copy(rows_v, out_hbm.at[pl.ds(base, _BPW)])


def _gather_rows(table, idx_pad):
    mesh = plsc.VectorSubcoreMesh(
        core_axis_name="c", subcore_axis_name="s",
        num_cores=_NC, num_subcores=_NS)
    k = pl.kernel(
        _sc_gather_body,
        out_type=jax.ShapeDtypeStruct((Q_PAD, D), jnp.float32),
        mesh=mesh,
        scratch_types=[
            pltpu.VMEM((_BPW,), jnp.int32),
            pltpu.VMEM((_BPW, D), jnp.float32),
            pltpu.SemaphoreType.DMA,
        ],
    )
    return k(table, idx_pad)


def _sc_cand_body(flat_hbm, idx_hbm, out_hbm, idx_v, got_v, sem):
    wid = lax.axis_index("s") * _NC + lax.axis_index("c")
    base = wid * CAND_PW
    pltpu.sync_copy(idx_hbm.at[pl.ds(base, CAND_PW)], idx_v)

    def fire(i, c):
        pltpu.make_async_copy(
            flat_hbm.at[idx_v.at[pl.ds(i * CAND_BATCH, CAND_BATCH)]],
            got_v.at[pl.ds(i * CAND_BATCH, CAND_BATCH)], sem).start()
        return c

    lax.fori_loop(0, CAND_NB, fire, 0)

    def drain(i, c):
        pltpu.make_async_copy(
            flat_hbm.at[idx_v.at[pl.ds(i * CAND_BATCH, CAND_BATCH)]],
            got_v.at[pl.ds(i * CAND_BATCH, CAND_BATCH)], sem).wait()
        return c

    lax.fori_loop(0, CAND_NB, drain, 0)
    pltpu.sync_copy(got_v, out_hbm.at[pl.ds(base, CAND_PW)])


def _gather_cands(cos_flat, idx_flat):
    mesh = plsc.VectorSubcoreMesh(
        core_axis_name="c", subcore_axis_name="s",
        num_cores=_NC, num_subcores=_NS)
    k = pl.kernel(
        _sc_cand_body,
        out_type=jax.ShapeDtypeStruct((CAND_TOT,), jnp.float32),
        mesh=mesh,
        scratch_types=[
            pltpu.VMEM((CAND_PW,), jnp.int32),
            pltpu.VMEM((CAND_PW,), jnp.float32),
            pltpu.SemaphoreType.DMA,
        ],
    )
    return k(cos_flat, idx_flat)


def _fgsm_pert(in3, dg3):
    absg = jnp.broadcast_to(
        jnp.sum(jnp.abs(dg3), axis=-1, keepdims=True), (B, L, D))
    l_iota = lax.broadcasted_iota(jnp.int32, (B, L, D), 1)
    rank = jnp.zeros((B, L, D), jnp.int32)
    for m in range(L):
        am = absg[:, m:m + 1, :]
        beats = (am > absg) | ((am == absg) & (m < l_iota))
        rank = rank + beats.astype(jnp.int32)
    mask3 = rank < K_SEL
    cnt = float(B * K_SEL * D)
    g1 = jnp.where(mask3, dg3, 0.0)
    mean = jnp.sum(g1) / cnt
    var = jnp.sum(jnp.where(mask3, (dg3 - mean) ** 2, 0.0)) / (cnt - 1.0)
    std = jnp.sqrt(var)
    lower = mean - std * STD_MULT
    upper = mean + std * STD_MULT
    mask2 = mask3 & ((dg3 < lower) | (dg3 > upper))
    sign = jnp.sign(jnp.where(mask2, dg3, 0.0))
    pert3 = in3 + EPS * sign
    return pert3.reshape(Q, D)


def _phase1_body(t_ref, in_ref, dg_ref, cos_ref, flat_ref, cols_ref,
                 pert_ref, qn_ref, rm_ref):
    i = pl.program_id(0)

    @pl.when(i == 0)
    def _init():
        pert = _fgsm_pert(in_ref[...], dg_ref[...])
        pert_ref[...] = pert
        qn_ref[...] = jnp.sqrt(jnp.sum(pert * pert, axis=1, keepdims=True))

    tt = t_ref[...]
    pert = pert_ref[...]
    s = lax.dot_general(pert, tt, (((1,), (0,)), ((), ())),
                        preferred_element_type=jnp.float32)
    tn = jnp.sqrt(jnp.sum(tt * tt, axis=0, keepdims=True))
    denom = jnp.maximum(qn_ref[...] * tn, 1e-8)
    cos = s / denom

    @pl.when(i == N_TILES - 1)
    def _mask_tail():
        col = i * V_TILE + lax.broadcasted_iota(jnp.int32, (Q, V_TILE), 1)
        cos_ref[...] = jnp.where(col < V, cos, NEG_INF)

    @pl.when(i < N_TILES - 1)
    def _store_full():
        cos_ref[...] = cos

    cosm = cos_ref[...]

    @pl.when(i == 0)
    def _rm_init():
        rm_ref[...] = cosm

    @pl.when(i > 0)
    def _rm_update():
        rm_ref[...] = jnp.maximum(rm_ref[...], cosm)

    @pl.when(i == N_TILES - 1)
    def _extract():
        cl_iota = lax.broadcasted_iota(jnp.int32, (Q, V_TILE), 1)
        cm = rm_ref[...]
        sel = []
        for _ in range(N_CHUNK):
            m = jnp.max(cm, axis=1, keepdims=True)
            mi = jnp.min(jnp.where(cm == m, cl_iota, BIG_I32),
                         axis=1, keepdims=True)
            sel.append(mi)
            cm = jnp.where(cl_iota == mi, NEG_INF, cm)
        t_off = lax.broadcasted_iota(jnp.int32, (Q, N_TILES), 1) * V_TILE
        cols = jnp.concatenate([c + t_off for c in sel], axis=1)
        r_iota = lax.broadcasted_iota(jnp.int32, (Q, N_CAND), 0)
        flat_ref[...] = r_iota * V_PAD + cols
        cols_ref[...] = cols


def _phase1(table, in3, dg3):
    return pl.pallas_call(
        _phase1_body,
        grid=(N_TILES,),
        in_specs=[
            pl.BlockSpec((D, V_TILE), lambda i: (0, i)),
            pl.BlockSpec((B, L, D), lambda i: (0, 0, 0)),
            pl.BlockSpec((B, L, D), lambda i: (0, 0, 0)),
        ],
        out_specs=[
            pl.BlockSpec((Q, V_TILE), lambda i: (0, i)),
            pl.BlockSpec((Q, N_CAND), lambda i: (0, 0)),
            pl.BlockSpec((Q, N_CAND), lambda i: (0, 0)),
        ],
        out_shape=[
            jax.ShapeDtypeStruct((Q, V_PAD), jnp.float32),
            jax.ShapeDtypeStruct((Q, N_CAND), jnp.int32),
            jax.ShapeDtypeStruct((Q, N_CAND), jnp.int32),
        ],
        scratch_shapes=[
            pltpu.VMEM((Q, D), jnp.float32),
            pltpu.VMEM((Q, 1), jnp.float32),
            pltpu.VMEM((Q, V_TILE), jnp.float32),
        ],
    )(table, in3, dg3)


def _phase2_body(cv_ref, ci_ref, vals_ref, idx_ref):
    cv = cv_ref[...]
    ci = ci_ref[...]
    ov, oi = [], []
    for _ in range(KNN_K):
        m = jnp.max(cv, axis=1, keepdims=True)
        mi = jnp.min(jnp.where(cv == m, ci, BIG_I32), axis=1, keepdims=True)
        ov.append(m)
        oi.append(mi)
        cv = jnp.where((cv == m) & (ci == mi), NEG_INF, cv)
    vals_ref[...] = jnp.concatenate(ov, axis=1)
    idx_ref[...] = jnp.concatenate(oi, axis=1)


def _phase2(cand_vals, cand_cols):
    return pl.pallas_call(
        _phase2_body,
        out_shape=[
            jax.ShapeDtypeStruct((Q, KNN_K), jnp.float32),
            jax.ShapeDtypeStruct((Q, KNN_K), jnp.int32),
        ],
    )(cand_vals, cand_cols)


def kernel(utterance, table, data_grad):
    idx = utterance.reshape(-1).astype(jnp.int32)
    idx_pad = jnp.concatenate([idx, jnp.zeros((Q_PAD - Q,), jnp.int32)])
    rows = _gather_rows(table, idx_pad)
    in3 = rows[:Q].reshape(B, L, D)
    tT = jnp.pad(table, ((0, V_PAD - V), (0, 0))).T
    cos_full, cand_flat, cand_cols = _phase1(tT, in3, data_grad)
    flat_idx = jnp.concatenate(
        [cand_flat.reshape(-1),
         jnp.zeros((CAND_TOT - Q * N_CAND,), jnp.int32)])
    got = _gather_cands(cos_full.reshape(-1), flat_idx)
    cand_vals = got[:Q * N_CAND].reshape(Q, N_CAND)
    vals, idx8 = _phase2(cand_vals, cand_cols)
    return vals.reshape(B, L, KNN_K), idx8.reshape(B, L, KNN_K)


# speedup vs baseline: 4.6435x; 1.0918x over previous
import jax
import jax.numpy as jnp
from jax import lax
from jax.experimental import pallas as pl
from jax.experimental.pallas import tpu as pltpu
from jax.experimental.pallas import tpu_sc as plsc

B = 16
L = 50
V = 100000
D = 128
EPS = 0.4
K_SEL = 25
STD_MULT = 3.0
KNN_K = 8

Q = B * L
V_TILE = 2048
N_TILES = -(-V // V_TILE)
V_PAD = N_TILES * V_TILE

N_CHUNK = 9
N_CAND = N_CHUNK * N_TILES

NEG_INF = float("-inf")
BIG_I32 = 2147483647

_NC = 2
_NS = 16
_NW = _NC * _NS
Q_PAD = 1024
_BPW = Q_PAD // _NW

CAND_BATCH = 128
CAND_TOT = 356352
CAND_PW = CAND_TOT // _NW
CAND_NB = CAND_PW // CAND_BATCH


def _sc_gather_body(table_hbm, idx_hbm, out_hbm, idx_v, rows_v, sem):
    wid = lax.axis_index("s") * _NC + lax.axis_index("c")
    base = wid * _BPW
    pltpu.sync_copy(idx_hbm.at[pl.ds(base, _BPW)], idx_v)
    pltpu.async_copy(table_hbm.at[idx_v], rows_v, sem).wait()
    pltpu.sync_copy(rows_v, out_hbm.at[pl.ds(base, _BPW)])


def _gather_rows(table, idx_pad):
    mesh = plsc.VectorSubcoreMesh(
        core_axis_name="c", subcore_axis_name="s",
        num_cores=_NC, num_subcores=_NS)
    k = pl.kernel(
        _sc_gather_body,
        out_type=jax.ShapeDtypeStruct((Q_PAD, D), jnp.float32),
        mesh=mesh,
        scratch_types=[
            pltpu.VMEM((_BPW,), jnp.int32),
            pltpu.VMEM((_BPW, D), jnp.float32),
            pltpu.SemaphoreType.DMA,
        ],
    )
    return k(table, idx_pad)


def _sc_cand_body(flat_hbm, idx_hbm, out_hbm, idx_v, got_v, sem):
    wid = lax.axis_index("s") * _NC + lax.axis_index("c")
    base = wid * CAND_PW
    pltpu.sync_copy(idx_hbm.at[pl.ds(base, CAND_PW)], idx_v)

    def fire(i, c):
        pltpu.make_async_copy(
            flat_hbm.at[idx_v.at[pl.ds(i * CAND_BATCH, CAND_BATCH)]],
            got_v.at[pl.ds(i * CAND_BATCH, CAND_BATCH)], sem).start()
        return c

    lax.fori_loop(0, CAND_NB, fire, 0)

    def drain(i, c):
        pltpu.make_async_copy(
            flat_hbm.at[idx_v.at[pl.ds(i * CAND_BATCH, CAND_BATCH)]],
            got_v.at[pl.ds(i * CAND_BATCH, CAND_BATCH)], sem).wait()
        return c

    lax.fori_loop(0, CAND_NB, drain, 0)
    pltpu.sync_copy(got_v, out_hbm.at[pl.ds(base, CAND_PW)])


def _gather_cands(cos_flat, idx_flat):
    mesh = plsc.VectorSubcoreMesh(
        core_axis_name="c", subcore_axis_name="s",
        num_cores=_NC, num_subcores=_NS)
    k = pl.kernel(
        _sc_cand_body,
        out_type=jax.ShapeDtypeStruct((CAND_TOT,), jnp.float32),
        mesh=mesh,
        scratch_types=[
            pltpu.VMEM((CAND_PW,), jnp.int32),
            pltpu.VMEM((CAND_PW,), jnp.float32),
            pltpu.SemaphoreType.DMA,
        ],
    )
    return k(cos_flat, idx_flat)


def _fgsm_pert(in3, dg3):
    absg = jnp.broadcast_to(
        jnp.sum(jnp.abs(dg3), axis=-1, keepdims=True), (B, L, D))
    l_iota = lax.broadcasted_iota(jnp.int32, (B, L, D), 1)
    rank = jnp.zeros((B, L, D), jnp.int32)
    for m in range(L):
        am = absg[:, m:m + 1, :]
        beats = (am > absg) | ((am == absg) & (m < l_iota))
        rank = rank + beats.astype(jnp.int32)
    mask3 = rank < K_SEL
    cnt = float(B * K_SEL * D)
    g1 = jnp.where(mask3, dg3, 0.0)
    mean = jnp.sum(g1) / cnt
    var = jnp.sum(jnp.where(mask3, (dg3 - mean) ** 2, 0.0)) / (cnt - 1.0)
    std = jnp.sqrt(var)
    lower = mean - std * STD_MULT
    upper = mean + std * STD_MULT
    mask2 = mask3 & ((dg3 < lower) | (dg3 > upper))
    sign = jnp.sign(jnp.where(mask2, dg3, 0.0))
    pert3 = in3 + EPS * sign
    return pert3.reshape(Q, D)


def _phase1_body(t_ref, in_ref, dg_ref, cos_ref, flat_ref, cols_ref,
                 pert_ref, qn_ref, rm_ref):
    i = pl.program_id(0)

    @pl.when(i == 0)
    def _init():
        pert = _fgsm_pert(in_ref[...], dg_ref[...])
        pert_ref[...] = pert
        qn_ref[...] = jnp.sqrt(jnp.sum(pert * pert, axis=1, keepdims=True))

    tt = t_ref[...]
    pert = pert_ref[...]
    s = lax.dot_general(pert, tt, (((1,), (1,)), ((), ())),
                        preferred_element_type=jnp.float32)
    tn_col = jnp.sqrt(jnp.sum(tt * tt, axis=1, keepdims=True))
    tn = tn_col.T
    denom = jnp.maximum(qn_ref[...] * tn, 1e-8)
    cos = s / denom

    @pl.when(i == N_TILES - 1)
    def _mask_tail():
        col = i * V_TILE + lax.broadcasted_iota(jnp.int32, (Q, V_TILE), 1)
        cos_ref[...] = jnp.where(col < V, cos, NEG_INF)

    @pl.when(i < N_TILES - 1)
    def _store_full():
        cos_ref[...] = cos

    cosm = cos_ref[...]

    @pl.when(i == 0)
    def _rm_init():
        rm_ref[...] = cosm

    @pl.when(i > 0)
    def _rm_update():
        rm_ref[...] = jnp.maximum(rm_ref[...], cosm)

    @pl.when(i == N_TILES - 1)
    def _extract():
        cl_iota = lax.broadcasted_iota(jnp.int32, (Q, V_TILE), 1)
        cm = rm_ref[...]
        sel = []
        for _ in range(N_CHUNK):
            m = jnp.max(cm, axis=1, keepdims=True)
            mi = jnp.min(jnp.where(cm == m, cl_iota, BIG_I32),
                         axis=1, keepdims=True)
            sel.append(mi)
            cm = jnp.where(cl_iota == mi, NEG_INF, cm)
        t_off = lax.broadcasted_iota(jnp.int32, (Q, N_TILES), 1) * V_TILE
        cols = jnp.concatenate([c + t_off for c in sel], axis=1)
        r_iota = lax.broadcasted_iota(jnp.int32, (Q, N_CAND), 0)
        flat_ref[...] = r_iota * V_PAD + cols
        cols_ref[...] = cols


def _phase1(table, in3, dg3):
    return pl.pallas_call(
        _phase1_body,
        grid=(N_TILES,),
        in_specs=[
            pl.BlockSpec((V_TILE, D), lambda i: (i, 0)),
            pl.BlockSpec((B, L, D), lambda i: (0, 0, 0)),
            pl.BlockSpec((B, L, D), lambda i: (0, 0, 0)),
        ],
        out_specs=[
            pl.BlockSpec((Q, V_TILE), lambda i: (0, i)),
            pl.BlockSpec((Q, N_CAND), lambda i: (0, 0)),
            pl.BlockSpec((Q, N_CAND), lambda i: (0, 0)),
        ],
        out_shape=[
            jax.ShapeDtypeStruct((Q, V_PAD), jnp.float32),
            jax.ShapeDtypeStruct((Q, N_CAND), jnp.int32),
            jax.ShapeDtypeStruct((Q, N_CAND), jnp.int32),
        ],
        scratch_shapes=[
            pltpu.VMEM((Q, D), jnp.float32),
            pltpu.VMEM((Q, 1), jnp.float32),
            pltpu.VMEM((Q, V_TILE), jnp.float32),
        ],
    )(table, in3, dg3)


def _phase2_body(cv_ref, ci_ref, vals_ref, idx_ref):
    cv = cv_ref[...]
    ci = ci_ref[...]
    ov, oi = [], []
    for _ in range(KNN_K):
        m = jnp.max(cv, axis=1, keepdims=True)
        mi = jnp.min(jnp.where(cv == m, ci, BIG_I32), axis=1, keepdims=True)
        ov.append(m)
        oi.append(mi)
        cv = jnp.where((cv == m) & (ci == mi), NEG_INF, cv)
    vals_ref[...] = jnp.concatenate(ov, axis=1)
    idx_ref[...] = jnp.concatenate(oi, axis=1)


def _phase2(cand_vals, cand_cols):
    return pl.pallas_call(
        _phase2_body,
        out_shape=[
            jax.ShapeDtypeStruct((Q, KNN_K), jnp.float32),
            jax.ShapeDtypeStruct((Q, KNN_K), jnp.int32),
        ],
    )(cand_vals, cand_cols)


def kernel(utterance, table, data_grad):
    idx = utterance.reshape(-1).astype(jnp.int32)
    idx_pad = jnp.concatenate([idx, jnp.zeros((Q_PAD - Q,), jnp.int32)])
    rows = _gather_rows(table, idx_pad)
    in3 = rows[:Q].reshape(B, L, D)
    cos_full, cand_flat, cand_cols = _phase1(table, in3, data_grad)
    flat_idx = jnp.concatenate(
        [cand_flat.reshape(-1),
         jnp.zeros((CAND_TOT - Q * N_CAND,), jnp.int32)])
    got = _gather_cands(cos_full.reshape(-1), flat_idx)
    cand_vals = got[:Q * N_CAND].reshape(Q, N_CAND)
    vals, idx8 = _phase2(cand_vals, cand_cols)
    return vals.reshape(B, L, KNN_K), idx8.reshape(B, L, KNN_K)


# speedup vs baseline: 8.5804x; 1.8478x over previous
import jax
import jax.numpy as jnp
from jax import lax
from jax.experimental import pallas as pl
from jax.experimental.pallas import tpu as pltpu
from jax.experimental.pallas import tpu_sc as plsc

B = 16
L = 50
V = 100000
D = 128
EPS = 0.4
K_SEL = 25
STD_MULT = 3.0
KNN_K = 8

Q = B * L
V_TILE = 2048
N_TILES = -(-V // V_TILE)
V_PAD = N_TILES * V_TILE
CPT = V_TILE // 128
N_CHUNKS = V_PAD // 128

N_SEL = 9
N_CAND = N_SEL * 128

NEG_INF = float("-inf")
BIG_I32 = 2147483647

_NC = 2
_NS = 16
_NW = _NC * _NS

Q_PAD = 1024
SEL_TOT = 7424


def _make_sc_row_gather(n_rows, n_idx):
    per_w = n_idx // _NW
    mesh = plsc.VectorSubcoreMesh(
        core_axis_name="c", subcore_axis_name="s",
        num_cores=_NC, num_subcores=_NS)

    def body(table_hbm, idx_hbm, out_hbm, idx_v, rows_v, sem):
        wid = lax.axis_index("s") * _NC + lax.axis_index("c")
        base = wid * per_w
        pltpu.sync_copy(idx_hbm.at[pl.ds(base, per_w)], idx_v)
        pltpu.async_copy(table_hbm.at[idx_v], rows_v, sem).wait()
        pltpu.sync_copy(rows_v, out_hbm.at[pl.ds(base, per_w)])

    return pl.kernel(
        body,
        out_type=jax.ShapeDtypeStruct((n_idx, D), jnp.float32),
        mesh=mesh,
        scratch_types=[
            pltpu.VMEM((per_w,), jnp.int32),
            pltpu.VMEM((per_w, D), jnp.float32),
            pltpu.SemaphoreType.DMA,
        ],
    )


def _fgsm_pert(in3, dg3):
    absg = jnp.broadcast_to(
        jnp.sum(jnp.abs(dg3), axis=-1, keepdims=True), (B, L, D))
    l_iota = lax.broadcasted_iota(jnp.int32, (B, L, D), 1)
    rank = jnp.zeros((B, L, D), jnp.int32)
    for m in range(L):
        am = absg[:, m:m + 1, :]
        beats = (am > absg) | ((am == absg) & (m < l_iota))
        rank = rank + beats.astype(jnp.int32)
    mask3 = rank < K_SEL
    cnt = float(B * K_SEL * D)
    g1 = jnp.where(mask3, dg3, 0.0)
    mean = jnp.sum(g1) / cnt
    var = jnp.sum(jnp.where(mask3, (dg3 - mean) ** 2, 0.0)) / (cnt - 1.0)
    std = jnp.sqrt(var)
    lower = mean - std * STD_MULT
    upper = mean + std * STD_MULT
    mask2 = mask3 & ((dg3 < lower) | (dg3 > upper))
    sign = jnp.sign(jnp.where(mask2, dg3, 0.0))
    pert3 = in3 + EPS * sign
    return pert3.reshape(Q, D)


def _phase1_body(t_ref, in_ref, dg_ref, cos_ref, selg_ref, gidx_ref,
                 pert_ref, qn_ref, cm_ref):
    i = pl.program_id(0)

    @pl.when(i == 0)
    def _init():
        pert = _fgsm_pert(in_ref[...], dg_ref[...])
        pert_ref[...] = pert
        qn_ref[...] = jnp.sqrt(jnp.sum(pert * pert, axis=1, keepdims=True))

    tt = t_ref[...]
    pert = pert_ref[...]
    s = lax.dot_general(pert, tt, (((1,), (1,)), ((), ())),
                        preferred_element_type=jnp.float32)
    tn_col = jnp.sqrt(jnp.sum(tt * tt, axis=1, keepdims=True))
    tn = tn_col.T
    denom = jnp.maximum(qn_ref[...] * tn, 1e-8)
    cos = s / denom

    def _emit(c):
        cms = []
        for j in range(CPT):
            cj = c[:, j * 128:(j + 1) * 128]
            cos_ref[j, :, :] = cj
            cms.append(jnp.max(cj, axis=1, keepdims=True))
        cm_ref[i] = jnp.concatenate(cms, axis=1)

    @pl.when(i == N_TILES - 1)
    def _mask_tail():
        col = i * V_TILE + lax.broadcasted_iota(jnp.int32, (Q, V_TILE), 1)
        _emit(jnp.where(col < V, cos, NEG_INF))

    @pl.when(i < N_TILES - 1)
    def _not_tail():
        _emit(cos)

    @pl.when(i == N_TILES - 1)
    def _extract():
        g_iota = lax.broadcasted_iota(jnp.int32, (Q, N_CHUNKS), 1)
        cm = jnp.concatenate([cm_ref[t] for t in range(N_TILES)], axis=1)
        sel = []
        for _ in range(N_SEL):
            m = jnp.max(cm, axis=1, keepdims=True)
            mi = jnp.min(jnp.where(cm == m, g_iota, BIG_I32),
                         axis=1, keepdims=True)
            sel.append(mi)
            cm = jnp.where(g_iota == mi, NEG_INF, cm)
        selg = jnp.concatenate(sel, axis=1)
        r_iota = lax.broadcasted_iota(jnp.int32, (Q, N_SEL), 0)
        selg_ref[...] = selg
        gidx_ref[...] = selg * Q + r_iota


def _phase1(table, in3, dg3):
    return pl.pallas_call(
        _phase1_body,
        grid=(N_TILES,),
        in_specs=[
            pl.BlockSpec((V_TILE, D), lambda i: (i, 0)),
            pl.BlockSpec((B, L, D), lambda i: (0, 0, 0)),
            pl.BlockSpec((B, L, D), lambda i: (0, 0, 0)),
        ],
        out_specs=[
            pl.BlockSpec((CPT, Q, 128), lambda i: (i, 0, 0)),
            pl.BlockSpec((Q, N_SEL), lambda i: (0, 0)),
            pl.BlockSpec((Q, N_SEL), lambda i: (0, 0)),
        ],
        out_shape=[
            jax.ShapeDtypeStruct((N_CHUNKS, Q, 128), jnp.float32),
            jax.ShapeDtypeStruct((Q, N_SEL), jnp.int32),
            jax.ShapeDtypeStruct((Q, N_SEL), jnp.int32),
        ],
        scratch_shapes=[
            pltpu.VMEM((Q, D), jnp.float32),
            pltpu.VMEM((Q, 1), jnp.float32),
            pltpu.VMEM((N_TILES, Q, CPT), jnp.float32),
        ],
    )(table, in3, dg3)


def _phase2_body(cand_ref, selg_ref, vals_ref, idx_ref):
    cv = cand_ref[...]
    selg = selg_ref[...]
    l_iota = lax.broadcasted_iota(jnp.int32, (Q, 128), 1)
    ci = jnp.concatenate(
        [selg[:, k:k + 1] * 128 + l_iota for k in range(N_SEL)], axis=1)
    ov, oi = [], []
    for _ in range(KNN_K):
        m = jnp.max(cv, axis=1, keepdims=True)
        mi = jnp.min(jnp.where(cv == m, ci, BIG_I32), axis=1, keepdims=True)
        ov.append(m)
        oi.append(mi)
        cv = jnp.where((cv == m) & (ci == mi), NEG_INF, cv)
    vals_ref[...] = jnp.concatenate(ov, axis=1)
    idx_ref[...] = jnp.concatenate(oi, axis=1)


def _phase2(cand, selg):
    return pl.pallas_call(
        _phase2_body,
        out_shape=[
            jax.ShapeDtypeStruct((Q, KNN_K), jnp.float32),
            jax.ShapeDtypeStruct((Q, KNN_K), jnp.int32),
        ],
    )(cand, selg)


def kernel(utterance, table, data_grad):
    idx = utterance.reshape(-1).astype(jnp.int32)
    idx_pad = jnp.concatenate([idx, jnp.zeros((Q_PAD - Q,), jnp.int32)])
    rows = _make_sc_row_gather(V, Q_PAD)(table, idx_pad)
    in3 = rows[:Q].reshape(B, L, D)
    cos3, selg, gidx = _phase1(table, in3, data_grad)
    cos_rows = cos3.reshape(N_CHUNKS * Q, 128)
    gidx_pad = jnp.concatenate(
        [gidx.reshape(-1), jnp.zeros((SEL_TOT - Q * N_SEL,), jnp.int32)])
    got = _make_sc_row_gather(N_CHUNKS * Q, SEL_TOT)(cos_rows, gidx_pad)
    cand = got[:Q * N_SEL].reshape(Q, N_CAND)
    vals, idx8 = _phase2(cand, selg)
    return vals.reshape(B, L, KNN_K), idx8.reshape(B, L, KNN_K)


# speedup vs baseline: 10.2648x; 1.1963x over previous
import jax
import jax.numpy as jnp
from jax import lax
from jax.experimental import pallas as pl
from jax.experimental.pallas import tpu as pltpu
from jax.experimental.pallas import tpu_sc as plsc

B = 16
L = 50
V = 100000
D = 128
EPS = 0.4
K_SEL = 25
STD_MULT = 3.0
KNN_K = 8

Q = B * L
V_TILE = 2048
N_TILES = -(-V // V_TILE)
V_PAD = N_TILES * V_TILE
CPT = V_TILE // 128
N_CHUNKS = V_PAD // 128

N_SEL = 9
N_CAND = N_SEL * 128

NEG_INF = float("-inf")
BIG_I32 = 2147483647

_NC = 2
_NS = 16
_NW = _NC * _NS

Q_PAD = 1024
SEL_TOT = 7424


def _make_sc_row_gather(n_rows, n_idx):
    per_w = n_idx // _NW
    mesh = plsc.VectorSubcoreMesh(
        core_axis_name="c", subcore_axis_name="s",
        num_cores=_NC, num_subcores=_NS)

    def body(table_hbm, idx_hbm, out_hbm, idx_v, rows_v, sem):
        wid = lax.axis_index("s") * _NC + lax.axis_index("c")
        base = wid * per_w
        pltpu.sync_copy(idx_hbm.at[pl.ds(base, per_w)], idx_v)
        pltpu.async_copy(table_hbm.at[idx_v], rows_v, sem).wait()
        pltpu.sync_copy(rows_v, out_hbm.at[pl.ds(base, per_w)])

    return pl.kernel(
        body,
        out_type=jax.ShapeDtypeStruct((n_idx, D), jnp.float32),
        mesh=mesh,
        scratch_types=[
            pltpu.VMEM((per_w,), jnp.int32),
            pltpu.VMEM((per_w, D), jnp.float32),
            pltpu.SemaphoreType.DMA,
        ],
    )


def _fgsm_pert(in3, dg3):
    absg = jnp.broadcast_to(
        jnp.sum(jnp.abs(dg3), axis=-1, keepdims=True), (B, L, D))
    l_iota = lax.broadcasted_iota(jnp.int32, (B, L, D), 1)
    rank = jnp.zeros((B, L, D), jnp.int32)
    for m in range(L):
        am = absg[:, m:m + 1, :]
        beats = (am > absg) | ((am == absg) & (m < l_iota))
        rank = rank + beats.astype(jnp.int32)
    mask3 = rank < K_SEL
    cnt = float(B * K_SEL * D)
    g1 = jnp.where(mask3, dg3, 0.0)
    mean = jnp.sum(g1) / cnt
    var = jnp.sum(jnp.where(mask3, (dg3 - mean) ** 2, 0.0)) / (cnt - 1.0)
    std = jnp.sqrt(var)
    lower = mean - std * STD_MULT
    upper = mean + std * STD_MULT
    mask2 = mask3 & ((dg3 < lower) | (dg3 > upper))
    sign = jnp.sign(jnp.where(mask2, dg3, 0.0))
    pert3 = in3 + EPS * sign
    return pert3.reshape(Q, D)


def _phase1_body(t_ref, in_ref, dg_ref, cos_ref, selg_ref, gidx_ref,
                 pert_ref, qn_ref, cm_ref):
    i = pl.program_id(0)

    @pl.when(i == 0)
    def _init():
        pert = _fgsm_pert(in_ref[...], dg_ref[...])
        pert_ref[...] = pert
        qn_ref[...] = jnp.sqrt(jnp.sum(pert * pert, axis=1, keepdims=True))

    tt = t_ref[...]
    pert = pert_ref[...]
    qn = qn_ref[...]
    tn_col = jnp.sqrt(jnp.sum(tt * tt, axis=1, keepdims=True))
    tn = tn_col.T

    def _emit(tail):
        l_iota = lax.broadcasted_iota(jnp.int32, (Q, 128), 1)
        cms = []
        for j in range(CPT):
            ttj = tt[j * 128:(j + 1) * 128, :]
            sj = lax.dot_general(pert, ttj, (((1,), (1,)), ((), ())),
                                 preferred_element_type=jnp.float32)
            dj = jnp.maximum(qn * tn[:, j * 128:(j + 1) * 128], 1e-8)
            cj = sj / dj
            if tail:
                col = i * V_TILE + j * 128 + l_iota
                cj = jnp.where(col < V, cj, NEG_INF)
            cos_ref[j, :, :] = cj
            cms.append(jnp.max(cj, axis=1, keepdims=True))
        cm_ref[i] = jnp.concatenate(cms, axis=1)

    @pl.when(i == N_TILES - 1)
    def _mask_tail():
        _emit(True)

    @pl.when(i < N_TILES - 1)
    def _not_tail():
        _emit(False)

    @pl.when(i == N_TILES - 1)
    def _extract():
        g_iota = lax.broadcasted_iota(jnp.int32, (Q, N_CHUNKS), 1)
        cm = jnp.concatenate([cm_ref[t] for t in range(N_TILES)], axis=1)
        sel = []
        for _ in range(N_SEL):
            m = jnp.max(cm, axis=1, keepdims=True)
            mi = jnp.min(jnp.where(cm == m, g_iota, BIG_I32),
                         axis=1, keepdims=True)
            sel.append(mi)
            cm = jnp.where(g_iota == mi, NEG_INF, cm)
        selg = jnp.concatenate(sel, axis=1)
        r_iota = lax.broadcasted_iota(jnp.int32, (Q, N_SEL), 0)
        selg_ref[...] = selg
        gidx_ref[...] = selg * Q + r_iota


def _phase1(table, in3, dg3):
    return pl.pallas_call(
        _phase1_body,
        grid=(N_TILES,),
        in_specs=[
            pl.BlockSpec((V_TILE, D), lambda i: (i, 0)),
            pl.BlockSpec((B, L, D), lambda i: (0, 0, 0)),
            pl.BlockSpec((B, L, D), lambda i: (0, 0, 0)),
        ],
        out_specs=[
            pl.BlockSpec((CPT, Q, 128), lambda i: (i, 0, 0)),
            pl.BlockSpec((Q, N_SEL), lambda i: (0, 0)),
            pl.BlockSpec((Q, N_SEL), lambda i: (0, 0)),
        ],
        out_shape=[
            jax.ShapeDtypeStruct((N_CHUNKS, Q, 128), jnp.float32),
            jax.ShapeDtypeStruct((Q, N_SEL), jnp.int32),
            jax.ShapeDtypeStruct((Q, N_SEL), jnp.int32),
        ],
        scratch_shapes=[
            pltpu.VMEM((Q, D), jnp.float32),
            pltpu.VMEM((Q, 1), jnp.float32),
            pltpu.VMEM((N_TILES, Q, CPT), jnp.float32),
        ],
    )(table, in3, dg3)


def _phase2_body(cand_ref, selg_ref, vals_ref, idx_ref):
    cv = cand_ref[...]
    selg = selg_ref[...]
    l_iota = lax.broadcasted_iota(jnp.int32, (Q, 128), 1)
    ci = jnp.concatenate(
        [selg[:, k:k + 1] * 128 + l_iota for k in range(N_SEL)], axis=1)
    ov, oi = [], []
    for _ in range(KNN_K):
        m = jnp.max(cv, axis=1, keepdims=True)
        mi = jnp.min(jnp.where(cv == m, ci, BIG_I32), axis=1, keepdims=True)
        ov.append(m)
        oi.append(mi)
        cv = jnp.where((cv == m) & (ci == mi), NEG_INF, cv)
    vals_ref[...] = jnp.concatenate(ov, axis=1)
    idx_ref[...] = jnp.concatenate(oi, axis=1)


def _phase2(cand, selg):
    return pl.pallas_call(
        _phase2_body,
        out_shape=[
            jax.ShapeDtypeStruct((Q, KNN_K), jnp.float32),
            jax.ShapeDtypeStruct((Q, KNN_K), jnp.int32),
        ],
    )(cand, selg)


def kernel(utterance, table, data_grad):
    idx = utterance.reshape(-1).astype(jnp.int32)
    idx_pad = jnp.concatenate([idx, jnp.zeros((Q_PAD - Q,), jnp.int32)])
    rows = _make_sc_row_gather(V, Q_PAD)(table, idx_pad)
    in3 = rows[:Q].reshape(B, L, D)
    cos3, selg, gidx = _phase1(table, in3, data_grad)
    cos_rows = cos3.reshape(N_CHUNKS * Q, 128)
    gidx_pad = jnp.concatenate(
        [gidx.reshape(-1), jnp.zeros((SEL_TOT - Q * N_SEL,), jnp.int32)])
    got = _make_sc_row_gather(N_CHUNKS * Q, SEL_TOT)(cos_rows, gidx_pad)
    cand = got[:Q * N_SEL].reshape(Q, N_CAND)
    vals, idx8 = _phase2(cand, selg)
    return vals.reshape(B, L, KNN_K), idx8.reshape(B, L, KNN_K)


# speedup vs baseline: 10.4755x; 1.0205x over previous
import jax
import jax.numpy as jnp
from jax import lax
from jax.experimental import pallas as pl
from jax.experimental.pallas import tpu as pltpu
from jax.experimental.pallas import tpu_sc as plsc

B = 16
L = 50
V = 100000
D = 128
EPS = 0.4
K_SEL = 25
STD_MULT = 3.0
KNN_K = 8

Q = B * L
V_TILE = 4096
N_TILES = -(-V // V_TILE)
V_PAD = N_TILES * V_TILE
CPT = V_TILE // 128
N_CHUNKS = V_PAD // 128

N_SEL = 9
N_CAND = N_SEL * 128

NEG_INF = float("-inf")
BIG_I32 = 2147483647

_NC = 2
_NS = 16
_NW = _NC * _NS

Q_PAD = 1024
SEL_TOT = 7424


def _make_sc_row_gather(n_rows, n_idx):
    per_w = n_idx // _NW
    mesh = plsc.VectorSubcoreMesh(
        core_axis_name="c", subcore_axis_name="s",
        num_cores=_NC, num_subcores=_NS)

    def body(table_hbm, idx_hbm, out_hbm, idx_v, rows_v, sem):
        wid = lax.axis_index("s") * _NC + lax.axis_index("c")
        base = wid * per_w
        pltpu.sync_copy(idx_hbm.at[pl.ds(base, per_w)], idx_v)
        pltpu.async_copy(table_hbm.at[idx_v], rows_v, sem).wait()
        pltpu.sync_copy(rows_v, out_hbm.at[pl.ds(base, per_w)])

    return pl.kernel(
        body,
        out_type=jax.ShapeDtypeStruct((n_idx, D), jnp.float32),
        mesh=mesh,
        scratch_types=[
            pltpu.VMEM((per_w,), jnp.int32),
            pltpu.VMEM((per_w, D), jnp.float32),
            pltpu.SemaphoreType.DMA,
        ],
    )


def _fgsm_pert(in3, dg3):
    absg = jnp.broadcast_to(
        jnp.sum(jnp.abs(dg3), axis=-1, keepdims=True), (B, L, D))
    l_iota = lax.broadcasted_iota(jnp.int32, (B, L, D), 1)
    rank = jnp.zeros((B, L, D), jnp.int32)
    for m in range(L):
        am = absg[:, m:m + 1, :]
        beats = (am > absg) | ((am == absg) & (m < l_iota))
        rank = rank + beats.astype(jnp.int32)
    mask3 = rank < K_SEL
    cnt = float(B * K_SEL * D)
    g1 = jnp.where(mask3, dg3, 0.0)
    mean = jnp.sum(g1) / cnt
    var = jnp.sum(jnp.where(mask3, (dg3 - mean) ** 2, 0.0)) / (cnt - 1.0)
    std = jnp.sqrt(var)
    lower = mean - std * STD_MULT
    upper = mean + std * STD_MULT
    mask2 = mask3 & ((dg3 < lower) | (dg3 > upper))
    sign = jnp.sign(jnp.where(mask2, dg3, 0.0))
    pert3 = in3 + EPS * sign
    return pert3.reshape(Q, D)


def _phase1_body(t_ref, in_ref, dg_ref, cos_ref, selg_ref, gidx_ref,
                 pert_ref, qn_ref, cm_ref):
    i = pl.program_id(0)

    @pl.when(i == 0)
    def _init():
        pert = _fgsm_pert(in_ref[...], dg_ref[...])
        pert_ref[...] = pert
        qn_ref[...] = jnp.sqrt(jnp.sum(pert * pert, axis=1, keepdims=True))

    tt = t_ref[...]
    pert = pert_ref[...]
    qn = qn_ref[...]
    tn_col = jnp.sqrt(jnp.sum(tt * tt, axis=1, keepdims=True))
    tn = tn_col.T

    def _emit(tail):
        l_iota = lax.broadcasted_iota(jnp.int32, (Q, 128), 1)
        cms = []
        for j in range(CPT):
            ttj = tt[j * 128:(j + 1) * 128, :]
            sj = lax.dot_general(pert, ttj, (((1,), (1,)), ((), ())),
                                 preferred_element_type=jnp.float32)
            dj = jnp.maximum(qn * tn[:, j * 128:(j + 1) * 128], 1e-8)
            cj = sj / dj
            if tail:
                col = i * V_TILE + j * 128 + l_iota
                cj = jnp.where(col < V, cj, NEG_INF)
            cos_ref[j, :, :] = cj
            cms.append(jnp.max(cj, axis=1, keepdims=True))
        cm_ref[i] = jnp.concatenate(cms, axis=1)

    @pl.when(i == N_TILES - 1)
    def _mask_tail():
        _emit(True)

    @pl.when(i < N_TILES - 1)
    def _not_tail():
        _emit(False)

    @pl.when(i == N_TILES - 1)
    def _extract():
        g_iota = lax.broadcasted_iota(jnp.int32, (Q, N_CHUNKS), 1)
        cm = jnp.concatenate([cm_ref[t] for t in range(N_TILES)], axis=1)
        sel = []
        for _ in range(N_SEL):
            m = jnp.max(cm, axis=1, keepdims=True)
            mi = jnp.min(jnp.where(cm == m, g_iota, BIG_I32),
                         axis=1, keepdims=True)
            sel.append(mi)
            cm = jnp.where(g_iota == mi, NEG_INF, cm)
        selg = jnp.concatenate(sel, axis=1)
        r_iota = lax.broadcasted_iota(jnp.int32, (Q, N_SEL), 0)
        selg_ref[...] = selg
        gidx_ref[...] = selg * Q + r_iota


def _phase1(table, in3, dg3):
    return pl.pallas_call(
        _phase1_body,
        grid=(N_TILES,),
        in_specs=[
            pl.BlockSpec((V_TILE, D), lambda i: (i, 0)),
            pl.BlockSpec((B, L, D), lambda i: (0, 0, 0)),
            pl.BlockSpec((B, L, D), lambda i: (0, 0, 0)),
        ],
        out_specs=[
            pl.BlockSpec((CPT, Q, 128), lambda i: (i, 0, 0)),
            pl.BlockSpec((Q, N_SEL), lambda i: (0, 0)),
            pl.BlockSpec((Q, N_SEL), lambda i: (0, 0)),
        ],
        out_shape=[
            jax.ShapeDtypeStruct((N_CHUNKS, Q, 128), jnp.float32),
            jax.ShapeDtypeStruct((Q, N_SEL), jnp.int32),
            jax.ShapeDtypeStruct((Q, N_SEL), jnp.int32),
        ],
        scratch_shapes=[
            pltpu.VMEM((Q, D), jnp.float32),
            pltpu.VMEM((Q, 1), jnp.float32),
            pltpu.VMEM((N_TILES, Q, CPT), jnp.float32),
        ],
    )(table, in3, dg3)


def _phase2_body(cand_ref, selg_ref, vals_ref, idx_ref):
    cv = cand_ref[...]
    selg = selg_ref[...]
    l_iota = lax.broadcasted_iota(jnp.int32, (Q, 128), 1)
    ci = jnp.concatenate(
        [selg[:, k:k + 1] * 128 + l_iota for k in range(N_SEL)], axis=1)
    ov, oi = [], []
    for _ in range(KNN_K):
        m = jnp.max(cv, axis=1, keepdims=True)
        mi = jnp.min(jnp.where(cv == m, ci, BIG_I32), axis=1, keepdims=True)
        ov.append(m)
        oi.append(mi)
        cv = jnp.where((cv == m) & (ci == mi), NEG_INF, cv)
    vals_ref[...] = jnp.concatenate(ov, axis=1)
    idx_ref[...] = jnp.concatenate(oi, axis=1)


def _phase2(cand, selg):
    return pl.pallas_call(
        _phase2_body,
        out_shape=[
            jax.ShapeDtypeStruct((Q, KNN_K), jnp.float32),
            jax.ShapeDtypeStruct((Q, KNN_K), jnp.int32),
        ],
    )(cand, selg)


def kernel(utterance, table, data_grad):
    idx = utterance.reshape(-1).astype(jnp.int32)
    idx_pad = jnp.concatenate([idx, jnp.zeros((Q_PAD - Q,), jnp.int32)])
    rows = _make_sc_row_gather(V, Q_PAD)(table, idx_pad)
    in3 = rows[:Q].reshape(B, L, D)
    cos3, selg, gidx = _phase1(table, in3, data_grad)
    cos_rows = cos3.reshape(N_CHUNKS * Q, 128)
    gidx_pad = jnp.concatenate(
        [gidx.reshape(-1), jnp.zeros((SEL_TOT - Q * N_SEL,), jnp.int32)])
    got = _make_sc_row_gather(N_CHUNKS * Q, SEL_TOT)(cos_rows, gidx_pad)
    cand = got[:Q * N_SEL].reshape(Q, N_CAND)
    vals, idx8 = _phase2(cand, selg)
    return vals.reshape(B, L, KNN_K), idx8.reshape(B, L, KNN_K)
